# baseline (speedup 1.0000x reference)
import functools

import jax
import jax.numpy as jnp
from jax import lax
from jax.experimental import pallas as pl
from jax.experimental.pallas import tpu as pltpu
from jax.experimental.pallas import tpu_sc as plsc

VOCAB = 20000
EMBED = 128
OUT = 6
BATCH = 4096
SEQ = 200
LANES = 16
PACK = 128 // LANES
NWORK = 32
BPW = BATCH // NWORK
PROJ_BLK = 500

CB = 4
NCH = BPW // CB
CHROWS = CB * SEQ


def _proj_body(e_ref, w_ref, b_ref, o_ref):
    o_ref[...] = lax.dot_general(
        e_ref[...], w_ref[...],
        (((1,), (0,)), ((), ())),
        preferred_element_type=jnp.float32,
    ) + b_ref[...]


def _project(e_packed, w_big, b_big):
    n = VOCAB // PACK
    return pl.pallas_call(
        _proj_body,
        out_shape=jax.ShapeDtypeStruct((n, 128), jnp.float32),
    )(e_packed, w_big, b_big)


def _make_sc_pool():
    mesh = plsc.VectorSubcoreMesh(core_axis_name="c", subcore_axis_name="s")

    @functools.partial(
        pl.kernel,
        out_type=jax.ShapeDtypeStruct((BATCH, LANES), jnp.float32),
        mesh=mesh,
        compiler_params=pltpu.CompilerParams(use_tc_tiling_on_sc=False),
        scratch_types=[
            pltpu.VMEM((BPW * SEQ,), jnp.int32),
            pltpu.VMEM((2, CHROWS, LANES), jnp.float32),
            pltpu.VMEM((BPW, LANES), jnp.float32),
            pltpu.SemaphoreType.DMA,
            pltpu.SemaphoreType.DMA,
        ],
    )
    def pool(p_hbm, idx_hbm, out_hbm, idx_v, rows_v, ost_v, sem0, sem1):
        wid = lax.axis_index("c") * 16 + lax.axis_index("s")
        base = wid * BPW
        pltpu.sync_copy(idx_hbm.at[pl.ds(base * SEQ, BPW * SEQ)], idx_v)
        sems = (sem0, sem1)

        def issue(c, p):
            pltpu.async_copy(
                p_hbm.at[idx_v.at[pl.ds(c * CHROWS, CHROWS)]],
                rows_v.at[p], sems[p])

        def wait(p):
            pltpu.make_async_copy(
                p_hbm.at[idx_v.at[pl.ds(0, CHROWS)]],
                rows_v.at[p], sems[p]).wait()

        def reduce_chunk(c, p):
            for k in range(CB):
                accs = [rows_v[p, k * SEQ + t] for t in range(8)]
                for j in range(8, SEQ):
                    accs[j % 8] += rows_v[p, k * SEQ + j]
                acc = (((accs[0] + accs[1]) + (accs[2] + accs[3]))
                       + ((accs[4] + accs[5]) + (accs[6] + accs[7])))
                ost_v[c * CB + k] = acc * (1.0 / SEQ)

        issue(0, 0)
        issue(1, 1)

        def chunk_pair(h, carry):
            for parity in range(2):
                c = h * 2 + parity
                wait(parity)
                reduce_chunk(c, parity)
                issue(c + 2, parity)
            return carry

        lax.fori_loop(0, NCH // 2 - 1, chunk_pair, 0)
        for parity in range(2):
            wait(parity)
            reduce_chunk(NCH - 2 + parity, parity)

        pltpu.sync_copy(ost_v, out_hbm.at[pl.ds(base, BPW)])

    return pool


_sc_pool = _make_sc_pool()


def kernel(indices, embed_table, fc_weight, fc_bias):
    wp = jnp.zeros((LANES, EMBED), jnp.float32).at[:OUT].set(fc_weight)
    bp = jnp.zeros((1, LANES), jnp.float32).at[0, :OUT].set(fc_bias)
    w_big = jnp.kron(jnp.eye(PACK, dtype=jnp.float32), wp.T)
    b_big = jnp.tile(bp, (1, PACK))
    e_packed = embed_table.reshape(VOCAB // PACK, PACK * EMBED)
    p_packed = _project(e_packed, w_big, b_big)
    p = p_packed.reshape(VOCAB, LANES)
    out16 = _sc_pool(p, indices.reshape(-1))
    return out16[:, :OUT][:, None, :]


# speedup vs baseline: 1.2593x; 1.2593x over previous
import functools

import jax
import jax.numpy as jnp
from jax import lax
from jax.experimental import pallas as pl
from jax.experimental.pallas import tpu as pltpu
from jax.experimental.pallas import tpu_sc as plsc

VOCAB = 20000
EMBED = 128
OUT = 6
BATCH = 4096
SEQ = 200
SEQP = 256
LANES = 16
PACK = 128 // LANES
NWORK = 32
BPW = BATCH // NWORK

CB = 8
NCH = BPW // CB
CHROWS = CB * SEQ

RPW = 125
NRW = (VOCAB // PACK) // RPW

_MESH = plsc.VectorSubcoreMesh(core_axis_name="c", subcore_axis_name="s")
_SC_PARAMS = pltpu.CompilerParams(use_tc_tiling_on_sc=False)


def _proj_body(e_ref, w_ref, b_ref, o_ref):
    o_ref[...] = lax.dot_general(
        e_ref[...], w_ref[...],
        (((1,), (0,)), ((), ())),
        preferred_element_type=jnp.float32,
    ) + b_ref[...]


def _project(e_packed, w_big, b_big):
    return pl.pallas_call(
        _proj_body,
        out_shape=jax.ShapeDtypeStruct((VOCAB // PACK, 128), jnp.float32),
    )(e_packed, w_big, b_big)


@functools.partial(
    pl.kernel,
    out_type=jax.ShapeDtypeStruct((VOCAB, LANES), jnp.float32),
    mesh=_MESH,
    compiler_params=_SC_PARAMS,
    scratch_types=[
        pltpu.VMEM((RPW, 128), jnp.float32),
        pltpu.VMEM((RPW * PACK, LANES), jnp.float32),
    ],
)
def _sc_repack(p_hbm, out_hbm, in_v, out_v):
    wid = lax.axis_index("c") * 16 + lax.axis_index("s")

    @pl.when(wid < NRW)
    def _():
        r0 = wid * RPW
        pltpu.sync_copy(p_hbm.at[pl.ds(r0, RPW)], in_v)

        def row(r, carry):
            for t in range(PACK):
                out_v[r * PACK + t] = in_v[r, pl.ds(t * LANES, LANES)]
            return carry

        lax.fori_loop(0, RPW, row, 0)
        pltpu.sync_copy(out_v, out_hbm.at[pl.ds(r0 * PACK, RPW * PACK)])


@functools.partial(
    pl.kernel,
    out_type=jax.ShapeDtypeStruct((BATCH // PACK, 128), jnp.float32),
    mesh=_MESH,
    compiler_params=_SC_PARAMS,
    scratch_types=[
        pltpu.VMEM((BPW, SEQP), jnp.int32),
        pltpu.VMEM((BPW * SEQ + 8,), jnp.int32),
        pltpu.VMEM((2, CHROWS, LANES), jnp.float32),
        pltpu.VMEM((BPW // PACK, 128), jnp.float32),
        pltpu.SemaphoreType.DMA,
        pltpu.SemaphoreType.DMA,
    ],
)
def _sc_pool(p_hbm, idx_hbm, out_hbm, idxp_v, idx_v, rows_v, ost_v,
             sem0, sem1):
    wid = lax.axis_index("c") * 16 + lax.axis_index("s")
    base = wid * BPW
    pltpu.sync_copy(idx_hbm.at[pl.ds(base, BPW)], idxp_v)

    def compact(k, carry):
        for t in range(13):
            idx_v[pl.ds(k * SEQ + t * LANES, LANES)] = (
                idxp_v[k, pl.ds(t * LANES, LANES)])
        return carry

    lax.fori_loop(0, BPW, compact, 0)

    sems = (sem0, sem1)

    def issue(c, p):
        pltpu.async_copy(
            p_hbm.at[idx_v.at[pl.ds(c * CHROWS, CHROWS)]],
            rows_v.at[p], sems[p])

    def wait(p):
        pltpu.make_async_copy(
            p_hbm.at[idx_v.at[pl.ds(0, CHROWS)]],
            rows_v.at[p], sems[p]).wait()

    def reduce_chunk(c, p):
        for k in range(CB):
            def red(i, accs):
                r0 = k * SEQ + i * 8
                return tuple(accs[t] + rows_v[p, r0 + t] for t in range(8))

            accs = lax.fori_loop(
                0, SEQ // 8, red,
                tuple(jnp.zeros((LANES,), jnp.float32) for _ in range(8)))
            acc = (((accs[0] + accs[1]) + (accs[2] + accs[3]))
                   + ((accs[4] + accs[5]) + (accs[6] + accs[7])))
            ost_v[c, pl.ds(k * LANES, LANES)] = acc * (1.0 / SEQ)

    issue(0, 0)
    issue(1, 1)

    def chunk_pair(h, carry):
        for parity in range(2):
            c = h * 2 + parity
            wait(parity)
            reduce_chunk(c, parity)
            issue(c + 2, parity)
        return carry

    lax.fori_loop(0, NCH // 2 - 1, chunk_pair, 0)
    for parity in range(2):
        wait(parity)
        reduce_chunk(NCH - 2 + parity, parity)

    pltpu.sync_copy(ost_v, out_hbm.at[pl.ds(wid * (BPW // PACK), BPW // PACK)])


def kernel(indices, embed_table, fc_weight, fc_bias):
    wp = jnp.zeros((LANES, EMBED), jnp.float32).at[:OUT].set(fc_weight)
    bp = jnp.zeros((1, LANES), jnp.float32).at[0, :OUT].set(fc_bias)
    w_big = jnp.kron(jnp.eye(PACK, dtype=jnp.float32), wp.T)
    b_big = jnp.tile(bp, (1, PACK))
    e_packed = embed_table.reshape(VOCAB // PACK, PACK * EMBED)
    p_packed = _project(e_packed, w_big, b_big)
    p_lin = _sc_repack(p_packed)
    idx_pad = jnp.pad(indices, ((0, 0), (0, SEQP - SEQ)))
    out_packed = _sc_pool(p_lin, idx_pad)
    out16 = out_packed.reshape(BATCH, LANES)
    return out16[:, :OUT][:, None, :]


# speedup vs baseline: 1.4054x; 1.1160x over previous
import functools

import jax
import jax.numpy as jnp
from jax import lax
from jax.experimental import pallas as pl
from jax.experimental.pallas import tpu as pltpu
from jax.experimental.pallas import tpu_sc as plsc

VOCAB = 20000
EMBED = 128
OUT = 6
BATCH = 4096
SEQ = 200
LANES = 16
PACK = 128 // LANES
NWORK = 32
BPW = BATCH // NWORK

CB = 8
NCH = BPW // CB
CHROWS = CB * SEQ

RPW = 125
NRW = (VOCAB // PACK) // RPW

_MESH = plsc.VectorSubcoreMesh(core_axis_name="c", subcore_axis_name="s")
_SC_PARAMS = pltpu.CompilerParams(use_tc_tiling_on_sc=False)


def _proj_body(e_ref, w_ref, b_ref, o_ref):
    for k in range(PACK):
        y = lax.dot_general(
            e_ref[:, k, :], w_ref[...],
            (((1,), (1,)), ((), ())),
            preferred_element_type=jnp.float32,
        ) + b_ref[...]
        o_ref[:, pl.ds(k * LANES, LANES)] = y


def _project(e3, wp, bp):
    return pl.pallas_call(
        _proj_body,
        out_shape=jax.ShapeDtypeStruct((VOCAB // PACK, 128), jnp.float32),
    )(e3, wp, bp)


@functools.partial(
    pl.kernel,
    out_type=jax.ShapeDtypeStruct((VOCAB, LANES), jnp.float32),
    mesh=_MESH,
    compiler_params=_SC_PARAMS,
    scratch_types=[
        pltpu.VMEM((RPW, 128), jnp.float32),
        pltpu.VMEM((RPW * PACK, LANES), jnp.float32),
    ],
)
def _sc_repack(p_hbm, out_hbm, in_v, out_v):
    wid = lax.axis_index("c") * 16 + lax.axis_index("s")

    @pl.when(wid < NRW)
    def _():
        r0 = wid * RPW
        pltpu.sync_copy(p_hbm.at[pl.ds(r0, RPW)], in_v)

        def row(r, carry):
            for t in range(PACK):
                out_v[r * PACK + t] = in_v[r, pl.ds(t * LANES, LANES)]
            return carry

        lax.fori_loop(0, RPW, row, 0)
        pltpu.sync_copy(out_v, out_hbm.at[pl.ds(r0 * PACK, RPW * PACK)])


@functools.partial(
    pl.kernel,
    out_type=jax.ShapeDtypeStruct((BATCH // PACK, 128), jnp.float32),
    mesh=_MESH,
    compiler_params=_SC_PARAMS,
    scratch_types=[
        pltpu.VMEM((BPW * SEQ,), jnp.int32),
        pltpu.VMEM((2, CHROWS, LANES), jnp.float32),
        pltpu.VMEM((BPW // PACK, 128), jnp.float32),
        pltpu.SemaphoreType.DMA,
        pltpu.SemaphoreType.DMA,
    ],
)
def _sc_pool(p_hbm, idx_hbm, out_hbm, idx_v, rows_v, ost_v, sem0, sem1):
    wid = lax.axis_index("c") * 16 + lax.axis_index("s")
    base = wid * BPW
    pltpu.sync_copy(idx_hbm.at[pl.ds(base * SEQ, BPW * SEQ)], idx_v)
    sems = (sem0, sem1)

    def issue(c, p):
        pltpu.async_copy(
            p_hbm.at[idx_v.at[pl.ds(c * CHROWS, CHROWS)]],
            rows_v.at[p], sems[p])

    def wait(p):
        pltpu.make_async_copy(
            p_hbm.at[idx_v.at[pl.ds(0, CHROWS)]],
            rows_v.at[p], sems[p]).wait()

    def reduce_chunk(c, p):
        for k in range(CB):
            def red(i, accs):
                r0 = k * SEQ + i * 8
                return tuple(accs[t] + rows_v[p, r0 + t] for t in range(8))

            accs = lax.fori_loop(
                0, SEQ // 8, red,
                tuple(jnp.zeros((LANES,), jnp.float32) for _ in range(8)))
            acc = (((accs[0] + accs[1]) + (accs[2] + accs[3]))
                   + ((accs[4] + accs[5]) + (accs[6] + accs[7])))
            ost_v[c, pl.ds(k * LANES, LANES)] = acc * (1.0 / SEQ)

    issue(0, 0)
    issue(1, 1)

    def chunk_pair(h, carry):
        for parity in range(2):
            c = h * 2 + parity
            wait(parity)
            reduce_chunk(c, parity)
            issue(c + 2, parity)
        return carry

    lax.fori_loop(0, NCH // 2 - 1, chunk_pair, 0)
    for parity in range(2):
        wait(parity)
        reduce_chunk(NCH - 2 + parity, parity)

    pltpu.sync_copy(ost_v, out_hbm.at[pl.ds(wid * (BPW // PACK), BPW // PACK)])


def kernel(indices, embed_table, fc_weight, fc_bias):
    wp = jnp.zeros((LANES, EMBED), jnp.float32).at[:OUT].set(fc_weight)
    bp = jnp.zeros((1, LANES), jnp.float32).at[0, :OUT].set(fc_bias)
    e3 = embed_table.reshape(VOCAB // PACK, PACK, EMBED)
    p_packed = _project(e3, wp, bp)
    p_lin = _sc_repack(p_packed)
    out_packed = _sc_pool(p_lin, indices.reshape(-1))
    out16 = out_packed.reshape(BATCH, LANES)
    return out16[:, :OUT][:, None, :]


# speedup vs baseline: 1.4238x; 1.0130x over previous
import functools

import jax
import jax.numpy as jnp
from jax import lax
from jax.experimental import pallas as pl
from jax.experimental.pallas import tpu as pltpu
from jax.experimental.pallas import tpu_sc as plsc

VOCAB = 20000
EMBED = 128
OUT = 6
BATCH = 4096
SEQ = 200
LANES = 16
PACK = 128 // LANES
NWORK = 32
BPW = BATCH // NWORK

CB = 8
NCH = BPW // CB
CHROWS = CB * SEQ
NBUF = 3

RPW = 125
NRW = (VOCAB // PACK) // RPW

PROJ_BLK = 250

_MESH = plsc.VectorSubcoreMesh(core_axis_name="c", subcore_axis_name="s")
_SC_PARAMS = pltpu.CompilerParams(use_tc_tiling_on_sc=False)


def _proj_body(e_ref, w_ref, b_ref, o_ref):
    w = jnp.concatenate(
        [w_ref[...], jnp.zeros((LANES - OUT, EMBED), jnp.float32)], axis=0)
    b = jnp.concatenate(
        [b_ref[...], jnp.zeros((1, LANES - OUT), jnp.float32)], axis=1)
    for k in range(PACK):
        y = lax.dot_general(
            e_ref[:, k, :], w,
            (((1,), (1,)), ((), ())),
            preferred_element_type=jnp.float32,
        ) + b
        o_ref[:, pl.ds(k * LANES, LANES)] = y


def _project(e3, fc_weight, fc_bias2):
    n = VOCAB // PACK
    return pl.pallas_call(
        _proj_body,
        out_shape=jax.ShapeDtypeStruct((n, 128), jnp.float32),
    )(e3, fc_weight, fc_bias2)


@functools.partial(
    pl.kernel,
    out_type=jax.ShapeDtypeStruct((VOCAB, LANES), jnp.float32),
    mesh=_MESH,
    compiler_params=_SC_PARAMS,
    scratch_types=[
        pltpu.VMEM((RPW, 128), jnp.float32),
        pltpu.VMEM((RPW * PACK, LANES), jnp.float32),
    ],
)
def _sc_repack(p_hbm, out_hbm, in_v, out_v):
    wid = lax.axis_index("c") * 16 + lax.axis_index("s")

    @pl.when(wid < NRW)
    def _():
        r0 = wid * RPW
        pltpu.sync_copy(p_hbm.at[pl.ds(r0, RPW)], in_v)

        def row(r, carry):
            for t in range(PACK):
                out_v[r * PACK + t] = in_v[r, pl.ds(t * LANES, LANES)]
            return carry

        lax.fori_loop(0, RPW, row, 0)
        pltpu.sync_copy(out_v, out_hbm.at[pl.ds(r0 * PACK, RPW * PACK)])


@functools.partial(
    pl.kernel,
    out_type=jax.ShapeDtypeStruct((BATCH // PACK, 128), jnp.float32),
    mesh=_MESH,
    compiler_params=_SC_PARAMS,
    scratch_types=[
        pltpu.VMEM((BPW, SEQ), jnp.int32),
        pltpu.VMEM((NBUF, CHROWS, LANES), jnp.float32),
        pltpu.VMEM((BPW // PACK, 128), jnp.float32),
        pltpu.SemaphoreType.DMA,
        pltpu.SemaphoreType.DMA,
        pltpu.SemaphoreType.DMA,
    ],
)
def _sc_pool(p_hbm, idx_hbm, out_hbm, idx_v, rows_v, ost_v, *sems):
    wid = lax.axis_index("c") * 16 + lax.axis_index("s")
    base = wid * BPW
    pltpu.sync_copy(idx_hbm.at[pl.ds(base, BPW)], idx_v)

    def issue(c, p):
        for k in range(CB):
            pltpu.async_copy(
                p_hbm.at[idx_v.at[c * CB + k]],
                rows_v.at[p].at[pl.ds(k * SEQ, SEQ)], sems[p])

    def wait(p):
        pltpu.make_async_copy(
            p_hbm.at[pl.ds(0, CHROWS)], rows_v.at[p], sems[p]).wait()

    def reduce_chunk(c, p):
        for k in range(CB):
            def red(i, accs):
                r0 = k * SEQ + i * 16
                return tuple(
                    accs[t] + (rows_v[p, r0 + t] + rows_v[p, r0 + 8 + t])
                    for t in range(8))

            accs = lax.fori_loop(
                0, SEQ // 16, red,
                tuple(jnp.zeros((LANES,), jnp.float32) for _ in range(8)))
            r0 = k * SEQ + 192
            accs = tuple(accs[t] + rows_v[p, r0 + t] for t in range(8))
            acc = (((accs[0] + accs[1]) + (accs[2] + accs[3]))
                   + ((accs[4] + accs[5]) + (accs[6] + accs[7])))
            ost_v[c, pl.ds(k * LANES, LANES)] = acc * (1.0 / SEQ)

    for p in range(NBUF):
        issue(p, p)

    def ring(h, carry):
        for q in range(NBUF):
            c = h * NBUF + q
            wait(q)
            reduce_chunk(c, q)
            issue(c + NBUF, q)
        return carry

    lax.fori_loop(0, (NCH - NBUF - 1) // NBUF, ring, 0)
    c0 = ((NCH - NBUF - 1) // NBUF) * NBUF
    for c in range(c0, NCH):
        wait(c % NBUF)
        reduce_chunk(c, c % NBUF)
        if c + NBUF < NCH:
            issue(c + NBUF, c % NBUF)

    pltpu.sync_copy(ost_v, out_hbm.at[pl.ds(wid * (BPW // PACK), BPW // PACK)])


def kernel(indices, embed_table, fc_weight, fc_bias):
    e3 = embed_table.reshape(VOCAB // PACK, PACK, EMBED)
    p_packed = _project(e3, fc_weight, fc_bias.reshape(1, OUT))
    p_lin = _sc_repack(p_packed)
    out_packed = _sc_pool(p_lin, indices)
    out16 = out_packed.reshape(BATCH, LANES)
    return out16[:, :OUT][:, None, :]


# speedup vs baseline: 1.4522x; 1.0200x over previous
import functools

import jax
import jax.numpy as jnp
from jax import lax
from jax.experimental import pallas as pl
from jax.experimental.pallas import tpu as pltpu
from jax.experimental.pallas import tpu_sc as plsc

VOCAB = 20000
EMBED = 128
OUT = 6
BATCH = 4096
SEQ = 200
LANES = 16
PACK = 128 // LANES
NWORK = 32
BPW = BATCH // NWORK

CB = 8
NCH = BPW // CB
CHROWS = CB * SEQ
NBUF = 3

RPW = 125
NRW = (VOCAB // PACK) // RPW

PROJ_BLK = 250

_MESH = plsc.VectorSubcoreMesh(core_axis_name="c", subcore_axis_name="s")
_SC_PARAMS = pltpu.CompilerParams(use_tc_tiling_on_sc=False)


def _proj_body(e_ref, w_ref, b_ref, o_ref):
    w = jnp.concatenate(
        [w_ref[...], jnp.zeros((LANES - OUT, EMBED), jnp.float32)], axis=0)
    b = jnp.concatenate(
        [b_ref[...], jnp.zeros((1, LANES - OUT), jnp.float32)], axis=1)
    for k in range(PACK):
        y = lax.dot_general(
            e_ref[:, k, :], w,
            (((1,), (1,)), ((), ())),
            preferred_element_type=jnp.float32,
        ) + b
        o_ref[:, pl.ds(k * LANES, LANES)] = y


def _project(e3, fc_weight, fc_bias2):
    n = VOCAB // PACK
    return pl.pallas_call(
        _proj_body,
        out_shape=jax.ShapeDtypeStruct((n, 128), jnp.float32),
    )(e3, fc_weight, fc_bias2)


@functools.partial(
    pl.kernel,
    out_type=jax.ShapeDtypeStruct((VOCAB, LANES), jnp.float32),
    mesh=_MESH,
    compiler_params=_SC_PARAMS,
    scratch_types=[
        pltpu.VMEM((RPW, 128), jnp.float32),
        pltpu.VMEM((RPW * PACK, LANES), jnp.float32),
    ],
)
def _sc_repack(p_hbm, out_hbm, in_v, out_v):
    wid = lax.axis_index("c") * 16 + lax.axis_index("s")

    @pl.when(wid < NRW)
    def _():
        r0 = wid * RPW
        pltpu.sync_copy(p_hbm.at[pl.ds(r0, RPW)], in_v)

        def row(r, carry):
            for t in range(PACK):
                out_v[r * PACK + t] = in_v[r, pl.ds(t * LANES, LANES)]
            return carry

        lax.fori_loop(0, RPW, row, 0)
        pltpu.sync_copy(out_v, out_hbm.at[pl.ds(r0 * PACK, RPW * PACK)])


@functools.partial(
    pl.kernel,
    out_type=jax.ShapeDtypeStruct((BATCH // PACK, 128), jnp.float32),
    mesh=_MESH,
    compiler_params=_SC_PARAMS,
    scratch_types=[
        pltpu.VMEM((BPW * SEQ,), jnp.int32),
        pltpu.VMEM((NBUF, CHROWS, LANES), jnp.float32),
        pltpu.VMEM((BPW // PACK, 128), jnp.float32),
        pltpu.SemaphoreType.DMA,
        pltpu.SemaphoreType.DMA,
        pltpu.SemaphoreType.DMA,
    ],
)
def _sc_pool(p_hbm, idx_hbm, out_hbm, idx_v, rows_v, ost_v, *sems):
    wid = lax.axis_index("c") * 16 + lax.axis_index("s")
    base = wid * BPW
    pltpu.sync_copy(idx_hbm.at[pl.ds(base * SEQ, BPW * SEQ)], idx_v)

    def issue(c, p):
        pltpu.async_copy(
            p_hbm.at[idx_v.at[pl.ds(c * CHROWS, CHROWS)]],
            rows_v.at[p], sems[p])

    def wait(p):
        pltpu.make_async_copy(
            p_hbm.at[pl.ds(0, CHROWS)], rows_v.at[p], sems[p]).wait()

    def reduce_chunk(c, p):
        for k in range(CB):
            def red(i, accs):
                r0 = k * SEQ + i * 8
                return tuple(accs[t] + rows_v[p, r0 + t] for t in range(8))

            accs = lax.fori_loop(
                0, SEQ // 8, red,
                tuple(jnp.zeros((LANES,), jnp.float32) for _ in range(8)))
            acc = (((accs[0] + accs[1]) + (accs[2] + accs[3]))
                   + ((accs[4] + accs[5]) + (accs[6] + accs[7])))
            ost_v[c, pl.ds(k * LANES, LANES)] = acc * (1.0 / SEQ)

    for p in range(NBUF):
        issue(p, p)

    def ring(h, carry):
        for q in range(NBUF):
            c = h * NBUF + q
            wait(q)
            reduce_chunk(c, q)
            issue(c + NBUF, q)
        return carry

    lax.fori_loop(0, (NCH - NBUF - 1) // NBUF, ring, 0)
    c0 = ((NCH - NBUF - 1) // NBUF) * NBUF
    for c in range(c0, NCH):
        wait(c % NBUF)
        reduce_chunk(c, c % NBUF)
        if c + NBUF < NCH:
            issue(c + NBUF, c % NBUF)

    pltpu.sync_copy(ost_v, out_hbm.at[pl.ds(wid * (BPW // PACK), BPW // PACK)])


def kernel(indices, embed_table, fc_weight, fc_bias):
    e3 = embed_table.reshape(VOCAB // PACK, PACK, EMBED)
    p_packed = _project(e3, fc_weight, fc_bias.reshape(1, OUT))
    p_lin = _sc_repack(p_packed)
    out_packed = _sc_pool(p_lin, indices.reshape(-1))
    out16 = out_packed.reshape(BATCH, LANES)
    return out16[:, :OUT][:, None, :]
